# baseline (speedup 1.0000x reference)
import functools

import jax
import jax.numpy as jnp
from jax import lax
from jax.experimental import pallas as pl
from jax.experimental.pallas import tpu as pltpu
from jax.experimental.pallas import tpu_sc as plsc

_N = 10000
_E = 320000
_D = 128
_H = 64
_B = 64

_NC, _NS = 2, 16
_NW = _NC * _NS
_EW = _E // _NW
_IB = 80
_CH = 400
_NIB = _CH // _IB
_NCHUNK = _EW // _CH
_NPAD = 10240
_RPT = _NPAD // _NS

_mesh = plsc.VectorSubcoreMesh(
    core_axis_name="c", subcore_axis_name="s", num_cores=_NC, num_subcores=_NS)


def _ab_body(h_ref, w_ref, b_ref, a_ref, bb_ref):
    h = h_ref[...]
    w = w_ref[...]
    a_ref[...] = jnp.dot(h, w[:_D], preferred_element_type=jnp.float32) + b_ref[...]
    bb_ref[...] = jnp.dot(h, w[_D:], preferred_element_type=jnp.float32)


def _ab2_body(hp_ref, w_ref, b_ref, a_ref, bb_ref):
    h = hp_ref[0, :_N] + hp_ref[1, :_N]
    w = w_ref[...]
    a_ref[...] = jnp.dot(h, w[:_D], preferred_element_type=jnp.float32) + b_ref[...]
    bb_ref[...] = jnp.dot(h, w[_D:], preferred_element_type=jnp.float32)


_ab_shapes = (jax.ShapeDtypeStruct((_N, _H), jnp.float32),
              jax.ShapeDtypeStruct((_N, _H), jnp.float32))
_ab = pl.pallas_call(_ab_body, out_shape=_ab_shapes)
_ab2 = pl.pallas_call(_ab2_body, out_shape=_ab_shapes)

_MB = 8000


def _msg_body(g_ref, w_ref, b_ref, o_ref):
    o_ref[...] = jnp.maximum(
        jnp.dot(g_ref[...], w_ref[...], preferred_element_type=jnp.float32)
        + b_ref[...], 0.0)


_msg = pl.pallas_call(
    _msg_body,
    grid=(_E // _MB,),
    in_specs=[
        pl.BlockSpec((_MB, _H), lambda i: (i, 0)),
        pl.BlockSpec((_H, _D), lambda i: (0, 0)),
        pl.BlockSpec((1, _D), lambda i: (0, 0)),
    ],
    out_specs=pl.BlockSpec((_MB, _D), lambda i: (i, 0)),
    out_shape=jax.ShapeDtypeStruct((_E, _D), jnp.float32),
)


def _ro_body(hp_ref, gid_ref, wi_ref, bi_ref, wh_ref, bh_ref, wo_ref, bo_ref,
             o_ref):
    h = hp_ref[0, :_N] + hp_ref[1, :_N]
    gid = gid_ref[...]
    oh = (gid == lax.broadcasted_iota(jnp.int32, (_B, _N), 0)).astype(jnp.float32)
    mol = jnp.dot(oh, h, preferred_element_type=jnp.float32)
    o = jnp.maximum(
        jnp.dot(mol, wi_ref[...], preferred_element_type=jnp.float32)
        + bi_ref[...], 0.0)
    for i in range(wh_ref.shape[0]):
        o = jnp.maximum(
            jnp.dot(o, wh_ref[i], preferred_element_type=jnp.float32)
            + bh_ref[i], 0.0)
    o_ref[...] = jnp.dot(o, wo_ref[...], preferred_element_type=jnp.float32) \
        + bo_ref[...]


def _gather_body(a_hbm, b_hbm, dst_hbm, src_hbm, g_hbm,
                 di, si, bufa, bufb, gbuf, sema, semb):
    c = lax.axis_index("c")
    s = lax.axis_index("s")
    wid = c * _NS + s
    base_e = wid * _EW

    def chunk(k, carry):
        g = wid * _NCHUNK + k
        eoff = base_e + k * _CH
        pltpu.sync_copy(dst_hbm.at[g], di)
        pltpu.sync_copy(src_hbm.at[g], si)
        for j in range(_NIB):
            cpa = pltpu.async_copy(a_hbm.at[di.at[j]], bufa, sema)
            cpb = pltpu.async_copy(b_hbm.at[si.at[j]], bufb, semb)
            cpa.wait()
            cpb.wait()

            def row(i, carry2):
                for jj in range(_H // 16):
                    a = bufa[i, pl.ds(jj * 16, 16)]
                    b = bufb[i, pl.ds(jj * 16, 16)]
                    gbuf[j * _IB + i, pl.ds(jj * 16, 16)] = \
                        jnp.maximum(a + b, 0.0)
                return carry2

            lax.fori_loop(0, _IB, row, 0)
        pltpu.sync_copy(gbuf, g_hbm.at[pl.ds(eoff, _CH)])
        return carry

    lax.fori_loop(0, _NCHUNK, chunk, 0)


_gather = pl.kernel(
    _gather_body,
    out_type=jax.ShapeDtypeStruct((_E, _H), jnp.float32),
    mesh=_mesh,
    compiler_params=pltpu.CompilerParams(use_tc_tiling_on_sc=False),
    scratch_types=[
        pltpu.VMEM((_NIB, _IB), jnp.int32),
        pltpu.VMEM((_NIB, _IB), jnp.int32),
        pltpu.VMEM((_IB, _H), jnp.float32),
        pltpu.VMEM((_IB, _H), jnp.float32),
        pltpu.VMEM((_CH, _H), jnp.float32),
        pltpu.SemaphoreType.DMA,
        pltpu.SemaphoreType.DMA,
    ],
)


def _scatter_body(m_hbm, dst_hbm, out_hbm, di, mbuf, acc):
    c = lax.axis_index("c")
    s = lax.axis_index("s")
    wid = c * _NS + s

    def zrow(i, carry):
        for jj in range(_D // 16):
            mbuf[i, pl.ds(jj * 16, 16)] = jnp.zeros((16,), jnp.float32)
        return carry
    lax.fori_loop(0, _IB, zrow, 0)
    r0 = s * _RPT

    def zcp(i, carry):
        pltpu.sync_copy(mbuf, acc.at[pl.ds(r0 + i * _IB, _IB)])
        return carry
    lax.fori_loop(0, _RPT // _IB, zcp, 0)
    plsc.subcore_barrier()

    def chunk(k, carry):
        g = wid * _NCHUNK + k
        eoff = wid * _EW + k * _CH
        pltpu.sync_copy(dst_hbm.at[g], di)
        for j in range(_NIB):
            pltpu.sync_copy(m_hbm.at[pl.ds(eoff + j * _IB, _IB)], mbuf)
            pltpu.sync_copy(mbuf, acc.at[di.at[j]], add=True)
        return carry

    lax.fori_loop(0, _NCHUNK, chunk, 0)
    plsc.subcore_barrier()

    def dcp(i, carry):
        pltpu.sync_copy(acc.at[pl.ds(r0 + i * _IB, _IB)], mbuf)
        pltpu.sync_copy(mbuf, out_hbm.at[c, pl.ds(r0 + i * _IB, _IB)])
        return carry
    lax.fori_loop(0, _RPT // _IB, dcp, 0)


_scatter = pl.kernel(
    _scatter_body,
    out_type=jax.ShapeDtypeStruct((_NC, _NPAD, _D), jnp.float32),
    mesh=_mesh,
    scratch_types=[
        pltpu.VMEM((_NIB, _IB), jnp.int32),
        pltpu.VMEM((_IB, _D), jnp.float32),
        pltpu.VMEM_SHARED((_NPAD, _D), jnp.float32),
    ],
)


def kernel(x, edge_index, graph_ids, msg_W_in, msg_b_in, msg_W_out, msg_b_out,
           ro_W_in, ro_b_in, ro_W_h, ro_b_h, ro_W_out, ro_b_out):
    src = edge_index[0].reshape(_E // _CH, _NIB, _IB)
    dst = edge_index[1].reshape(_E // _CH, _NIB, _IB)
    gid = graph_ids.reshape(1, _N)

    hp = None
    for s in range(msg_W_in.shape[0]):
        w_in = msg_W_in[s]
        b_in = msg_b_in[s].reshape(1, _H)
        if hp is None:
            a, b = _ab(x, w_in, b_in)
        else:
            a, b = _ab2(hp, w_in, b_in)
        g = _gather(a, b, dst, src)
        m = _msg(g, msg_W_out[s], msg_b_out[s].reshape(1, _D))
        hp = _scatter(m, dst)

    nro = ro_W_h.shape[0]
    ro = pl.pallas_call(
        _ro_body,
        out_shape=jax.ShapeDtypeStruct((_B, ro_W_out.shape[1]), jnp.float32),
    )
    return ro(hp, gid, ro_W_in, ro_b_in.reshape(1, -1), ro_W_h,
              ro_b_h.reshape(nro, 1, -1), ro_W_out,
              ro_b_out.reshape(1, -1))


# speedup vs baseline: 1.3486x; 1.3486x over previous
import functools

import jax
import jax.numpy as jnp
from jax import lax
from jax.experimental import pallas as pl
from jax.experimental.pallas import tpu as pltpu
from jax.experimental.pallas import tpu_sc as plsc

_N = 10000
_E = 320000
_D = 128
_H = 64
_B = 64

_NC, _NS = 2, 16
_NW = _NC * _NS
_EW = _E // _NW
_IB = 80
_CH = 400
_NIB = _CH // _IB
_NCHUNK = _EW // _CH
_NPAD = 10240
_RPT = _NPAD // _NS

_mesh = plsc.VectorSubcoreMesh(
    core_axis_name="c", subcore_axis_name="s", num_cores=_NC, num_subcores=_NS)


def _ab_body(h_ref, w_ref, b_ref, t_ref):
    h = h_ref[...]
    w = w_ref[...]
    a = jnp.dot(h, w[:_D], preferred_element_type=jnp.float32) + b_ref[...]
    b = jnp.dot(h, w[_D:], preferred_element_type=jnp.float32)
    t_ref[...] = jnp.concatenate([a, b], axis=1)


def _ab2_body(hp_ref, w_ref, b_ref, t_ref):
    h = hp_ref[0, :_N] + hp_ref[1, :_N]
    w = w_ref[...]
    a = jnp.dot(h, w[:_D], preferred_element_type=jnp.float32) + b_ref[...]
    b = jnp.dot(h, w[_D:], preferred_element_type=jnp.float32)
    t_ref[...] = jnp.concatenate([a, b], axis=1)


_ab_shape = jax.ShapeDtypeStruct((_N, 2 * _H), jnp.float32)
_ab = pl.pallas_call(_ab_body, out_shape=_ab_shape)
_ab2 = pl.pallas_call(_ab2_body, out_shape=_ab_shape)

_MB = 8000


def _msg_body(g_ref, w_ref, b_ref, o_ref):
    o_ref[...] = jnp.maximum(
        jnp.dot(g_ref[...], w_ref[...], preferred_element_type=jnp.float32)
        + b_ref[...], 0.0)


_msg = pl.pallas_call(
    _msg_body,
    grid=(_E // _MB,),
    in_specs=[
        pl.BlockSpec((_MB, _H), lambda i: (i, 0)),
        pl.BlockSpec((_H, _D), lambda i: (0, 0)),
        pl.BlockSpec((1, _D), lambda i: (0, 0)),
    ],
    out_specs=pl.BlockSpec((_MB, _D), lambda i: (i, 0)),
    out_shape=jax.ShapeDtypeStruct((_E, _D), jnp.float32),
)


def _ro_body(hp_ref, gid_ref, wi_ref, bi_ref, wh_ref, bh_ref, wo_ref, bo_ref,
             o_ref):
    h = hp_ref[0, :_N] + hp_ref[1, :_N]
    gid = gid_ref[...]
    oh = (gid == lax.broadcasted_iota(jnp.int32, (_B, _N), 0)).astype(jnp.float32)
    mol = jnp.dot(oh, h, preferred_element_type=jnp.float32)
    o = jnp.maximum(
        jnp.dot(mol, wi_ref[...], preferred_element_type=jnp.float32)
        + bi_ref[...], 0.0)
    for i in range(wh_ref.shape[0]):
        o = jnp.maximum(
            jnp.dot(o, wh_ref[i], preferred_element_type=jnp.float32)
            + bh_ref[i], 0.0)
    o_ref[...] = jnp.dot(o, wo_ref[...], preferred_element_type=jnp.float32) \
        + bo_ref[...]


def _gather_body(t_hbm, dst_hbm, src_hbm, g_hbm,
                 di, si, ba0, ba1, bb0, bb1, go0, go1,
                 sa0, sa1, sb0, sb1, sg0, sg1):
    c = lax.axis_index("c")
    s = lax.axis_index("s")
    wid = c * _NS + s
    base_e = wid * _EW
    ba = (ba0, ba1)
    bb = (bb0, bb1)
    go = (go0, go1)
    sa = (sa0, sa1)
    sb = (sb0, sb1)
    sg = (sg0, sg1)

    def chunk(k, carry):
        g = wid * _NCHUNK + k
        eoff = base_e + k * _CH
        pltpu.sync_copy(dst_hbm.at[g], di)
        pltpu.sync_copy(src_hbm.at[g], si)
        pend_ab = {}
        pend_g = {}

        def fire(j):
            p = j % 2
            pend_ab[j] = (
                pltpu.async_copy(t_hbm.at[di.at[j]], ba[p], sa[p]),
                pltpu.async_copy(t_hbm.at[si.at[j]], bb[p], sb[p]))

        fire(0)
        for j in range(_NIB):
            if j + 1 < _NIB:
                fire(j + 1)
            ca, cb = pend_ab.pop(j)
            ca.wait()
            cb.wait()
            if j - 2 in pend_g:
                pend_g.pop(j - 2).wait()
            p = j % 2

            def row(i, carry2, p=p):
                for jj in range(_H // 16):
                    a = ba[p][i, pl.ds(jj * 16, 16)]
                    b = bb[p][i, pl.ds(_H + jj * 16, 16)]
                    go[p][i, pl.ds(jj * 16, 16)] = jnp.maximum(a + b, 0.0)
                return carry2

            lax.fori_loop(0, _IB, row, 0)
            pend_g[j] = pltpu.async_copy(
                go[p], g_hbm.at[pl.ds(eoff + j * _IB, _IB)], sg[p])
        for j in sorted(pend_g):
            pend_g.pop(j).wait()
        return carry

    lax.fori_loop(0, _NCHUNK, chunk, 0)


_gather = pl.kernel(
    _gather_body,
    out_type=jax.ShapeDtypeStruct((_E, _H), jnp.float32),
    mesh=_mesh,
    scratch_types=[
        pltpu.VMEM((_NIB, _IB), jnp.int32),
        pltpu.VMEM((_NIB, _IB), jnp.int32),
        pltpu.VMEM((_IB, 2 * _H), jnp.float32),
        pltpu.VMEM((_IB, 2 * _H), jnp.float32),
        pltpu.VMEM((_IB, 2 * _H), jnp.float32),
        pltpu.VMEM((_IB, 2 * _H), jnp.float32),
        pltpu.VMEM((_IB, _H), jnp.float32),
        pltpu.VMEM((_IB, _H), jnp.float32),
        pltpu.SemaphoreType.DMA,
        pltpu.SemaphoreType.DMA,
        pltpu.SemaphoreType.DMA,
        pltpu.SemaphoreType.DMA,
        pltpu.SemaphoreType.DMA,
        pltpu.SemaphoreType.DMA,
    ],
)


def _scatter_body(m_hbm, dst_hbm, out_hbm, di, mb0, mb1, acc,
                  sm0, sm1, ss0, ss1):
    c = lax.axis_index("c")
    s = lax.axis_index("s")
    wid = c * _NS + s
    mb = (mb0, mb1)
    sm = (sm0, sm1)
    ss = (ss0, ss1)

    def zrow(i, carry):
        for jj in range(_D // 16):
            mb0[i, pl.ds(jj * 16, 16)] = jnp.zeros((16,), jnp.float32)
        return carry
    lax.fori_loop(0, _IB, zrow, 0)
    r0 = s * _RPT

    def zcp(i, carry):
        pltpu.sync_copy(mb0, acc.at[pl.ds(r0 + i * _IB, _IB)])
        return carry
    lax.fori_loop(0, _RPT // _IB, zcp, 0)
    plsc.subcore_barrier()

    def chunk(k, carry):
        g = wid * _NCHUNK + k
        eoff = wid * _EW + k * _CH
        pltpu.sync_copy(dst_hbm.at[g], di)
        pend_m = {}
        pend_s = {}

        def fire(j):
            p = j % 2
            pend_m[j] = pltpu.async_copy(
                m_hbm.at[pl.ds(eoff + j * _IB, _IB)], mb[p], sm[p])

        fire(0)
        for j in range(_NIB):
            if j + 1 < _NIB:
                fire(j + 1)
            pend_m.pop(j).wait()
            if j - 2 in pend_s:
                pend_s.pop(j - 2).wait()
            p = j % 2
            pend_s[j] = pltpu.async_copy(mb[p], acc.at[di.at[j]], ss[p],
                                         add=True)
        for j in sorted(pend_s):
            pend_s.pop(j).wait()
        return carry

    lax.fori_loop(0, _NCHUNK, chunk, 0)
    plsc.subcore_barrier()

    def dcp(i, carry):
        pltpu.sync_copy(acc.at[pl.ds(r0 + i * _IB, _IB)], mb0)
        pltpu.sync_copy(mb0, out_hbm.at[c, pl.ds(r0 + i * _IB, _IB)])
        return carry
    lax.fori_loop(0, _RPT // _IB, dcp, 0)


_scatter = pl.kernel(
    _scatter_body,
    out_type=jax.ShapeDtypeStruct((_NC, _NPAD, _D), jnp.float32),
    mesh=_mesh,
    scratch_types=[
        pltpu.VMEM((_NIB, _IB), jnp.int32),
        pltpu.VMEM((_IB, _D), jnp.float32),
        pltpu.VMEM((_IB, _D), jnp.float32),
        pltpu.VMEM_SHARED((_NPAD, _D), jnp.float32),
        pltpu.SemaphoreType.DMA,
        pltpu.SemaphoreType.DMA,
        pltpu.SemaphoreType.DMA,
        pltpu.SemaphoreType.DMA,
    ],
)


def kernel(x, edge_index, graph_ids, msg_W_in, msg_b_in, msg_W_out, msg_b_out,
           ro_W_in, ro_b_in, ro_W_h, ro_b_h, ro_W_out, ro_b_out):
    src = edge_index[0].reshape(_E // _CH, _NIB, _IB)
    dst = edge_index[1].reshape(_E // _CH, _NIB, _IB)
    gid = graph_ids.reshape(1, _N)

    hp = None
    for s in range(msg_W_in.shape[0]):
        w_in = msg_W_in[s]
        b_in = msg_b_in[s].reshape(1, _H)
        if hp is None:
            t = _ab(x, w_in, b_in)
        else:
            t = _ab2(hp, w_in, b_in)
        g = _gather(t, dst, src)
        m = _msg(g, msg_W_out[s], msg_b_out[s].reshape(1, _D))
        hp = _scatter(m, dst)

    nro = ro_W_h.shape[0]
    ro = pl.pallas_call(
        _ro_body,
        out_shape=jax.ShapeDtypeStruct((_B, ro_W_out.shape[1]), jnp.float32),
    )
    return ro(hp, gid, ro_W_in, ro_b_in.reshape(1, -1), ro_W_h,
              ro_b_h.reshape(nro, 1, -1), ro_W_out,
              ro_b_out.reshape(1, -1))


# speedup vs baseline: 1.3743x; 1.0190x over previous
import jax
import jax.numpy as jnp
from jax import lax
from jax.experimental import pallas as pl
from jax.experimental.pallas import tpu as pltpu
from jax.experimental.pallas import tpu_sc as plsc

_N = 10000
_E = 320000
_D = 128
_H = 64
_B = 64

_E1 = 192000
_E2 = _E - _E1

_NC, _NS = 2, 16
_NW = _NC * _NS
_IB = 80
_CH = 400
_NIB = _CH // _IB
_NPAD = 10240
_RPT = _NPAD // _NS

_mesh = plsc.VectorSubcoreMesh(
    core_axis_name="c", subcore_axis_name="s", num_cores=_NC, num_subcores=_NS)


def _ab_body(h_ref, w_ref, b_ref, t_ref):
    h = h_ref[...]
    w = w_ref[...]
    a = jnp.dot(h, w[:_D], preferred_element_type=jnp.float32, precision=lax.Precision.HIGHEST) + b_ref[...]
    b = jnp.dot(h, w[_D:], preferred_element_type=jnp.float32, precision=lax.Precision.HIGHEST)
    t_ref[...] = jnp.concatenate([a, b], axis=1)


def _ab2_body(hpa_ref, hpb_ref, w_ref, b_ref, t_ref):
    h = (hpa_ref[0, :_N] + hpa_ref[1, :_N]
         + hpb_ref[0, :_N] + hpb_ref[1, :_N])
    w = w_ref[...]
    a = jnp.dot(h, w[:_D], preferred_element_type=jnp.float32, precision=lax.Precision.HIGHEST) + b_ref[...]
    b = jnp.dot(h, w[_D:], preferred_element_type=jnp.float32, precision=lax.Precision.HIGHEST)
    t_ref[...] = jnp.concatenate([a, b], axis=1)


_ab_shape = jax.ShapeDtypeStruct((_N, 2 * _H), jnp.float32)
_ab = pl.pallas_call(_ab_body, out_shape=_ab_shape)
_ab2 = pl.pallas_call(_ab2_body, out_shape=_ab_shape)

_MB = 8000


def _msg_body(g_ref, w_ref, b_ref, o_ref):
    o_ref[...] = jnp.maximum(
        jnp.dot(g_ref[...], w_ref[...], preferred_element_type=jnp.float32, precision=lax.Precision.HIGHEST)
        + b_ref[...], 0.0)


def _make_msg(ne):
    return pl.pallas_call(
        _msg_body,
        grid=(ne // _MB,),
        in_specs=[
            pl.BlockSpec((_MB, _H), lambda i: (i, 0)),
            pl.BlockSpec((_H, _D), lambda i: (0, 0)),
            pl.BlockSpec((1, _D), lambda i: (0, 0)),
        ],
        out_specs=pl.BlockSpec((_MB, _D), lambda i: (i, 0)),
        out_shape=jax.ShapeDtypeStruct((ne, _D), jnp.float32),
    )


_msg1 = _make_msg(_E1)
_msg2 = _make_msg(_E2)


def _ro_body(hpa_ref, hpb_ref, gid_ref, wi_ref, bi_ref, wh_ref, bh_ref,
             wo_ref, bo_ref, o_ref):
    h = (hpa_ref[0, :_N] + hpa_ref[1, :_N]
         + hpb_ref[0, :_N] + hpb_ref[1, :_N])
    gid = gid_ref[...]
    oh = (gid == lax.broadcasted_iota(jnp.int32, (_B, _N), 0)).astype(jnp.float32)
    mol = jnp.dot(oh, h, preferred_element_type=jnp.float32, precision=lax.Precision.HIGHEST)
    o = jnp.maximum(
        jnp.dot(mol, wi_ref[...], preferred_element_type=jnp.float32, precision=lax.Precision.HIGHEST)
        + bi_ref[...], 0.0)
    for i in range(wh_ref.shape[0]):
        o = jnp.maximum(
            jnp.dot(o, wh_ref[i], preferred_element_type=jnp.float32, precision=lax.Precision.HIGHEST)
            + bh_ref[i], 0.0)
    o_ref[...] = jnp.dot(o, wo_ref[...], preferred_element_type=jnp.float32, precision=lax.Precision.HIGHEST) \
        + bo_ref[...]


def _make_gather(ne):
    ew = ne // _NW
    nchunk = ew // _CH

    def body(dep_hbm, t_hbm, dst_hbm, src_hbm, g_hbm,
             di, si, ba0, ba1, bb0, bb1, go0, go1,
             sa0, sa1, sb0, sb1, sg0, sg1):
        del dep_hbm
        c = lax.axis_index("c")
        s = lax.axis_index("s")
        wid = c * _NS + s
        base_e = wid * ew
        ba = (ba0, ba1)
        bb = (bb0, bb1)
        go = (go0, go1)
        sa = (sa0, sa1)
        sb = (sb0, sb1)
        sg = (sg0, sg1)

        def chunk(k, carry):
            g = wid * nchunk + k
            eoff = base_e + k * _CH
            pltpu.sync_copy(dst_hbm.at[g], di)
            pltpu.sync_copy(src_hbm.at[g], si)
            pend_ab = {}
            pend_g = {}

            def fire(j):
                p = j % 2
                pend_ab[j] = (
                    pltpu.async_copy(t_hbm.at[di.at[j]], ba[p], sa[p]),
                    pltpu.async_copy(t_hbm.at[si.at[j]], bb[p], sb[p]))

            fire(0)
            for j in range(_NIB):
                if j + 1 < _NIB:
                    fire(j + 1)
                ca, cb = pend_ab.pop(j)
                ca.wait()
                cb.wait()
                if j - 2 in pend_g:
                    pend_g.pop(j - 2).wait()
                p = j % 2

                def row(i, carry2, p=p):
                    for jj in range(_H // 16):
                        a = ba[p][i, pl.ds(jj * 16, 16)]
                        b = bb[p][i, pl.ds(_H + jj * 16, 16)]
                        go[p][i, pl.ds(jj * 16, 16)] = jnp.maximum(a + b, 0.0)
                    return carry2

                lax.fori_loop(0, _IB, row, 0)
                pend_g[j] = pltpu.async_copy(
                    go[p], g_hbm.at[pl.ds(eoff + j * _IB, _IB)], sg[p])
            for j in sorted(pend_g):
                pend_g.pop(j).wait()
            return carry

        lax.fori_loop(0, nchunk, chunk, 0)

    return pl.kernel(
        body,
        out_type=jax.ShapeDtypeStruct((ne, _H), jnp.float32),
        mesh=_mesh,
        scratch_types=[
            pltpu.VMEM((_NIB, _IB), jnp.int32),
            pltpu.VMEM((_NIB, _IB), jnp.int32),
            pltpu.VMEM((_IB, 2 * _H), jnp.float32),
            pltpu.VMEM((_IB, 2 * _H), jnp.float32),
            pltpu.VMEM((_IB, 2 * _H), jnp.float32),
            pltpu.VMEM((_IB, 2 * _H), jnp.float32),
            pltpu.VMEM((_IB, _H), jnp.float32),
            pltpu.VMEM((_IB, _H), jnp.float32),
            pltpu.SemaphoreType.DMA,
            pltpu.SemaphoreType.DMA,
            pltpu.SemaphoreType.DMA,
            pltpu.SemaphoreType.DMA,
            pltpu.SemaphoreType.DMA,
            pltpu.SemaphoreType.DMA,
        ],
    )


_gather1 = _make_gather(_E1)
_gather2 = _make_gather(_E2)


def _make_scatter(ne):
    ew = ne // _NW
    nchunk = ew // _CH

    def body(dep_hbm, m_hbm, dst_hbm, out_hbm, di, mb0, mb1, acc,
             sm0, sm1, ss0, ss1):
        del dep_hbm
        c = lax.axis_index("c")
        s = lax.axis_index("s")
        wid = c * _NS + s
        mb = (mb0, mb1)
        sm = (sm0, sm1)
        ss = (ss0, ss1)

        def zrow(i, carry):
            for jj in range(_D // 16):
                mb0[i, pl.ds(jj * 16, 16)] = jnp.zeros((16,), jnp.float32)
            return carry
        lax.fori_loop(0, _IB, zrow, 0)
        r0 = s * _RPT

        def zcp(i, carry):
            pltpu.sync_copy(mb0, acc.at[pl.ds(r0 + i * _IB, _IB)])
            return carry
        lax.fori_loop(0, _RPT // _IB, zcp, 0)
        plsc.subcore_barrier()

        def chunk(k, carry):
            g = wid * nchunk + k
            eoff = wid * ew + k * _CH
            pltpu.sync_copy(dst_hbm.at[g], di)
            pend_m = {}
            pend_s = {}

            def fire(j):
                p = j % 2
                pend_m[j] = pltpu.async_copy(
                    m_hbm.at[pl.ds(eoff + j * _IB, _IB)], mb[p], sm[p])

            fire(0)
            for j in range(_NIB):
                if j + 1 < _NIB:
                    fire(j + 1)
                pend_m.pop(j).wait()
                if j - 2 in pend_s:
                    pend_s.pop(j - 2).wait()
                p = j % 2
                pend_s[j] = pltpu.async_copy(mb[p], acc.at[di.at[j]], ss[p],
                                             add=True)
            for j in sorted(pend_s):
                pend_s.pop(j).wait()
            return carry

        lax.fori_loop(0, nchunk, chunk, 0)
        plsc.subcore_barrier()

        def dcp(i, carry):
            pltpu.sync_copy(acc.at[pl.ds(r0 + i * _IB, _IB)], mb0)
            pltpu.sync_copy(mb0, out_hbm.at[c, pl.ds(r0 + i * _IB, _IB)])
            return carry
        lax.fori_loop(0, _RPT // _IB, dcp, 0)

    return pl.kernel(
        body,
        out_type=jax.ShapeDtypeStruct((_NC, _NPAD, _D), jnp.float32),
        mesh=_mesh,
        scratch_types=[
            pltpu.VMEM((_NIB, _IB), jnp.int32),
            pltpu.VMEM((_IB, _D), jnp.float32),
            pltpu.VMEM((_IB, _D), jnp.float32),
            pltpu.VMEM_SHARED((_NPAD, _D), jnp.float32),
            pltpu.SemaphoreType.DMA,
            pltpu.SemaphoreType.DMA,
            pltpu.SemaphoreType.DMA,
            pltpu.SemaphoreType.DMA,
        ],
    )


_scatter1 = _make_scatter(_E1)
_scatter2 = _make_scatter(_E2)


def kernel(x, edge_index, graph_ids, msg_W_in, msg_b_in, msg_W_out, msg_b_out,
           ro_W_in, ro_b_in, ro_W_h, ro_b_h, ro_W_out, ro_b_out):
    src1 = edge_index[0, :_E1].reshape(_E1 // _CH, _NIB, _IB)
    dst1 = edge_index[1, :_E1].reshape(_E1 // _CH, _NIB, _IB)
    src2 = edge_index[0, _E1:].reshape(_E2 // _CH, _NIB, _IB)
    dst2 = edge_index[1, _E1:].reshape(_E2 // _CH, _NIB, _IB)
    gid = graph_ids.reshape(1, _N)

    hpa = hpb = None
    for s in range(msg_W_in.shape[0]):
        w_in = msg_W_in[s]
        b_in = msg_b_in[s].reshape(1, _H)
        b_out = msg_b_out[s].reshape(1, _D)
        if hpa is None:
            t = _ab(x, w_in, b_in)
        else:
            t = _ab2(hpa, hpb, w_in, b_in)
        g1 = _gather1(t, t, dst1, src1)
        g2 = _gather2(g1, t, dst2, src2)
        m1 = _msg1(g1, msg_W_out[s], b_out)
        m2 = _msg2(g2, msg_W_out[s], b_out)
        hpa = _scatter1(g2, m1, dst1)
        hpb = _scatter2(hpa, m2, dst2)

    nro = ro_W_h.shape[0]
    ro = pl.pallas_call(
        _ro_body,
        out_shape=jax.ShapeDtypeStruct((_B, ro_W_out.shape[1]), jnp.float32),
    )
    return ro(hpa, hpb, gid, ro_W_in, ro_b_in.reshape(1, -1), ro_W_h,
              ro_b_h.reshape(nro, 1, -1), ro_W_out,
              ro_b_out.reshape(1, -1))
